# baseline (speedup 1.0000x reference)
import functools

import jax
import jax.numpy as jnp
from jax import lax
from jax.experimental import pallas as pl
from jax.experimental.pallas import tpu as pltpu
from jax.experimental.pallas import tpu_sc as plsc

_RADIUS = 2
_TOP_K = 5000
_NEG = -jnp.inf

_H = 384
_W = 384
_HW = _H * _W
_B = 4
_C = 96
_SPAN = _HW // 8
_SLOT = 2176
_NSLOT = 8 * _SLOT
_SENT = -1
_PUB = 5120
_PER_TILE = 640
_KPAD = 8 * _PER_TILE
_NCH = _PER_TILE // 16


def _mp5(x):
    h, w = x.shape
    negr = jnp.full((2, w), _NEG, x.dtype)
    xp = jnp.concatenate([negr, x, negr], axis=0)
    r = jnp.maximum(
        jnp.maximum(xp[0:h], xp[1 : h + 1]),
        jnp.maximum(xp[3 : h + 3], xp[4 : h + 4]),
    )
    r = jnp.maximum(r, x)
    negc = jnp.full((h, 2), _NEG, x.dtype)
    rp = jnp.concatenate([negc, r, negc], axis=1)
    out = jnp.maximum(
        jnp.maximum(rp[:, 0:w], rp[:, 1 : w + 1]),
        jnp.maximum(rp[:, 3 : w + 3], rp[:, 4 : w + 4]),
    )
    return jnp.maximum(out, r)


def _nms_body(s_ref, o_ref):
    scores = s_ref[0, 0]
    h, w = scores.shape
    zeros = jnp.zeros_like(scores)
    max_mask = scores == _mp5(scores)
    for _ in range(2):
        supp_mask = _mp5(max_mask.astype(scores.dtype)) > 0.0
        supp_scores = jnp.where(supp_mask, zeros, scores)
        new_max_mask = supp_scores == _mp5(supp_scores)
        max_mask = max_mask | (new_max_mask & (~supp_mask))
    nms = jnp.where(max_mask, scores, zeros)
    r = _RADIUS
    row = lax.broadcasted_iota(jnp.int32, (h, w), 0)
    col = lax.broadcasted_iota(jnp.int32, (h, w), 1)
    interior = (row > r) & (row < h - r) & (col > r) & (col < w - r)
    o_ref[0, 0] = jnp.where(interior, nms, zeros)


def _nms_map(scores_map):
    b, c, h, w = scores_map.shape
    return pl.pallas_call(
        _nms_body,
        out_shape=jax.ShapeDtypeStruct((b, c, h, w), scores_map.dtype),
        grid=(b,),
        in_specs=[pl.BlockSpec((1, 1, h, w), lambda i: (i, 0, 0, 0))],
        out_specs=pl.BlockSpec((1, 1, h, w), lambda i: (i, 0, 0, 0)),
    )(scores_map)


def _iota16():
    return lax.iota(jnp.int32, 16)


_GTR_DNUMS = lax.GatherDimensionNumbers(
    offset_dims=(), collapsed_slice_dims=(0,), start_index_map=(0,))


def _vperm(x, idx):
    return lax.gather(
        x, idx[:, None], dimension_numbers=_GTR_DNUMS, slice_sizes=(1,),
        mode=lax.GatherScatterMode.PROMISE_IN_BOUNDS)


def _dup_rank(d, m, rtmp):
    it = _iota16()
    d2 = jnp.where(m, d, 2048 + it)
    sk, sl = plsc.sort_key_val(d2, it)
    prev = _vperm(sk, jnp.maximum(it - 1, 0))
    boundary = jnp.where(sk != prev, it, 0)
    segstart = plsc.cummax(boundary)
    rank_sorted = it - segstart
    plsc.store_scatter(rtmp, [sl], rank_sorted)
    return rtmp[pl.ds(0, 16)]


def _radix_sort(sv, si, ov, oi, hist, rtmp, n_live):
    it = _iota16()
    zeros16 = jnp.zeros((16,), jnp.int32)
    ones16 = jnp.full((16,), 1, jnp.int32)

    def _pass(p, _):
        shift = jnp.where(p == 0, 0, jnp.where(p == 1, 11, 22))
        limit = jnp.where(p == 0, _NSLOT, n_live)

        def _clr(i, __):
            hist[pl.ds(i * 16, 16)] = zeros16
            return 0

        lax.fori_loop(0, 2048 // 16, _clr, 0)

        def _count(i, __):
            k = sv[pl.ds(i * 16, 16)]
            m = (k != _SENT) & ((i * 16 + it) < limit)

            @pl.when(jnp.any(m))
            def __():
                d = lax.shift_right_logical(k, shift) & 2047
                plsc.addupdate_scatter(hist, [d], ones16, mask=m)
            return 0

        lax.fori_loop(0, _NSLOT // 16, _count, 0)

        def _prefix(i, carry):
            c = hist[pl.ds(i * 16, 16)]
            incl = plsc.cumsum(c)
            hist[pl.ds(i * 16, 16)] = carry + (incl - c)
            return carry + incl[15]

        lax.fori_loop(0, 2048 // 16, _prefix, jnp.int32(0))

        def _scatter(i, __):
            k = sv[pl.ds(i * 16, 16)]
            m = (k != _SENT) & ((i * 16 + it) < limit)

            @pl.when(jnp.any(m))
            def __():
                v = si[pl.ds(i * 16, 16)]
                d = lax.shift_right_logical(k, shift) & 2047
                base = plsc.load_gather(hist, [d])
                pos = base + _dup_rank(d, m, rtmp)
                plsc.store_scatter(ov, [pos], k, mask=m)
                plsc.store_scatter(oi, [pos], v, mask=m)
                plsc.addupdate_scatter(hist, [d], ones16, mask=m)
            return 0

        lax.fori_loop(0, _NSLOT // 16, _scatter, 0)

        def _copyback(i, __):
            @pl.when(i * 16 < n_live)
            def __():
                sv[pl.ds(i * 16, 16)] = ov[pl.ds(i * 16, 16)]
                si[pl.ds(i * 16, 16)] = oi[pl.ds(i * 16, 16)]
            return 0

        lax.fori_loop(0, _NSLOT // 16, _copyback, 0)
        return 0

    lax.fori_loop(0, 3, _pass, 0)


def _newton_rsqrt(x):
    bits = lax.bitcast_convert_type(x, jnp.int32)
    y = lax.bitcast_convert_type(
        jnp.int32(0x5F3759DF) - lax.shift_right_logical(bits, 1), jnp.float32)
    for _ in range(3):
        y = y * (1.5 - 0.5 * x * y * y)
    return y


def _sc_body(nms_hbm, sco_hbm, des_hbm, kp_out, de_out, ks_out,
             nmsbuf, lock, loci, cntbuf, cnt_all,
             sv, si, ov, oi, hist, rtmp, sidx,
             scidx, sdst, didx, gbuf, outb, kpbuf, ksbuf,
             counts_sh, ck_sh, ci_sh, sidx_sh, sem, sem2):
    cid = lax.axis_index("c")
    sid = lax.axis_index("s")
    bl = sid // 8
    chunk = sid % 8
    batch = 2 * cid + bl
    it = _iota16()

    pltpu.sync_copy(nms_hbm.at[pl.ds(batch * _HW + chunk * _SPAN, _SPAN)],
                    nmsbuf)
    base_idx = chunk * _SPAN

    def _compact(i, ptr):
        v = nmsbuf[pl.ds(i * 16, 16)]
        m = (v > 0.0) & (ptr < _SLOT - 16)
        kv = ~lax.bitcast_convert_type(v, jnp.int32)
        idxv = base_idx + i * 16 + it
        mi = m.astype(jnp.int32)
        pref = plsc.cumsum(mi)
        pos = ptr + (pref - mi)
        plsc.store_scatter(lock, [pos], kv, mask=m)
        plsc.store_scatter(loci, [pos], idxv, mask=m)
        return ptr + pref[15]

    ptr = lax.fori_loop(0, _SPAN // 16, _compact, jnp.int32(0))

    def _pad(i, _):
        cur = lock[pl.ds(i * 16, 16)]
        m = (i * 16 + it) < ptr
        lock[pl.ds(i * 16, 16)] = jnp.where(m, cur, _SENT)
        return 0

    lax.fori_loop(0, _SLOT // 16, _pad, 0)

    slot = (bl * 8 + chunk) * _SLOT
    pltpu.sync_copy(lock.at[pl.ds(0, _SLOT)], ck_sh.at[pl.ds(slot, _SLOT)])
    pltpu.sync_copy(loci.at[pl.ds(0, _SLOT)], ci_sh.at[pl.ds(slot, _SLOT)])
    cntbuf[...] = jnp.broadcast_to(ptr, (16,))
    pltpu.sync_copy(cntbuf, counts_sh.at[pl.ds(sid * 16, 16)])
    plsc.subcore_barrier()

    pltpu.sync_copy(counts_sh, cnt_all)
    nv = jnp.zeros((16,), jnp.int32)
    for j in range(8):
        nv = nv + cnt_all[pl.ds((bl * 8 + j) * 16, 16)]
    n_live = nv[0]

    @pl.when(chunk == 0)
    def _sort():
        for j in range(8):
            src = (bl * 8 + j) * _SLOT
            pltpu.sync_copy(ck_sh.at[pl.ds(src, _SLOT)],
                            sv.at[pl.ds(j * _SLOT, _SLOT)])
            pltpu.sync_copy(ci_sh.at[pl.ds(src, _SLOT)],
                            si.at[pl.ds(j * _SLOT, _SLOT)])
        _radix_sort(sv, si, ov, oi, hist, rtmp, n_live)
        pltpu.sync_copy(si.at[pl.ds(0, _PUB)],
                        sidx_sh.at[pl.ds(bl * _PUB, _PUB)])

    plsc.subcore_barrier()

    myoff = chunk * _PER_TILE
    pltpu.sync_copy(sidx_sh.at[pl.ds(bl * _PUB + myoff, _PER_TILE)], sidx)

    kp_pos0 = 2 * it
    d_pos = it * _C

    def _emit(j, _):
        p = myoff + j * 16 + it
        sidx_v = sidx[pl.ds(j * 16, 16)]
        real = p < n_live
        idx = jnp.where(real, sidx_v, p - n_live)

        t = lax.shift_right_logical(idx, 7)
        ky = lax.shift_right_logical(t * 683, 11)
        kx = idx - ky * 384
        kp_x = kx.astype(jnp.float32)
        kp_y = ky.astype(jnp.float32)
        ax = kp_x / 383.0 * 2.0 - 1.0
        ay = kp_y / 383.0 * 2.0 - 1.0
        plsc.store_scatter(kpbuf, [kp_pos0 + j * 32], ax)
        plsc.store_scatter(kpbuf, [kp_pos0 + j * 32 + 1], ay)

        px = (ax + 1.0) / 2.0 * 383.0
        py = (ay + 1.0) / 2.0 * 383.0
        x0i0 = px.astype(jnp.int32)
        y0i0 = py.astype(jnp.int32)
        x0 = x0i0.astype(jnp.float32)
        y0 = y0i0.astype(jnp.float32)
        wx1 = px - x0
        wx0 = 1.0 - wx1
        wy1 = py - y0
        wy0 = 1.0 - wy1
        x0i = jnp.minimum(x0i0, 383)
        x1i = jnp.minimum(x0i0 + 1, 383)
        y0i = jnp.minimum(y0i0, 383)
        y1i = jnp.minimum(y0i0 + 1, 383)
        sbase = batch * _HW
        scidx[pl.ds(0, 16)] = sbase + y0i * 384 + x0i
        scidx[pl.ds(16, 16)] = sbase + y0i * 384 + x1i
        scidx[pl.ds(32, 16)] = sbase + y1i * 384 + x0i
        scidx[pl.ds(48, 16)] = sbase + y1i * 384 + x1i
        sc_cp = pltpu.async_copy(sco_hbm.at[scidx], sdst, sem2)

        kxi = px.astype(jnp.int32)
        kyi = py.astype(jnp.int32)
        dflat = batch * (_C * _HW) + kyi * 384 + kxi
        for r in range(12):
            for q in range(8):
                c = r * 8 + q
                didx[r, pl.ds(q * 16, 16)] = dflat + c * _HW
        copies = [pltpu.async_copy(des_hbm.at[didx.at[r]], gbuf.at[r], sem)
                  for r in range(12)]

        sc_cp.wait()
        v00 = sdst[pl.ds(0, 16)]
        v01 = sdst[pl.ds(16, 16)]
        v10 = sdst[pl.ds(32, 16)]
        v11 = sdst[pl.ds(48, 16)]
        ksbuf[pl.ds(j * 16, 16)] = (v00 * wy0 * wx0 + v01 * wy0 * wx1
                                    + v10 * wy1 * wx0 + v11 * wy1 * wx1)

        for cp in copies:
            cp.wait()
        acc = jnp.zeros((16,), jnp.float32)
        for r in range(12):
            for q in range(8):
                g = gbuf[r, pl.ds(q * 16, 16)]
                acc = acc + g * g
        rns = _newton_rsqrt(acc)
        for r in range(12):
            for q in range(8):
                c = r * 8 + q
                g = gbuf[r, pl.ds(q * 16, 16)]
                plsc.store_scatter(outb, [d_pos + c], g * rns)
        pltpu.sync_copy(
            outb,
            de_out.at[pl.ds(batch * (_KPAD * _C) + (myoff + j * 16) * _C,
                            16 * _C)])
        return 0

    lax.fori_loop(0, _NCH, _emit, 0)

    pltpu.sync_copy(
        kpbuf,
        kp_out.at[pl.ds(batch * 2 * _KPAD + 2 * myoff, 2 * _PER_TILE)])
    pltpu.sync_copy(
        ksbuf, ks_out.at[pl.ds(batch * _KPAD + myoff, _PER_TILE)])


def _sc_select(nms_flat, scores_flat, desc_flat):
    mesh = plsc.VectorSubcoreMesh(core_axis_name="c", subcore_axis_name="s")
    f32 = jnp.float32
    i32 = jnp.int32
    kern = pl.kernel(
        _sc_body,
        out_type=[
            jax.ShapeDtypeStruct((_B * 2 * _KPAD,), f32),
            jax.ShapeDtypeStruct((_B * _KPAD * _C,), f32),
            jax.ShapeDtypeStruct((_B * _KPAD,), f32),
        ],
        mesh=mesh,
        compiler_params=pltpu.CompilerParams(needs_layout_passes=False),
        scratch_types=[
            pltpu.VMEM((_SPAN,), f32),
            pltpu.VMEM((_SLOT + 16,), i32),
            pltpu.VMEM((_SLOT + 16,), i32),
            pltpu.VMEM((16,), i32),
            pltpu.VMEM((256,), i32),
            pltpu.VMEM((_NSLOT,), i32),
            pltpu.VMEM((_NSLOT,), i32),
            pltpu.VMEM((_NSLOT,), i32),
            pltpu.VMEM((_NSLOT,), i32),
            pltpu.VMEM((2048,), i32),
            pltpu.VMEM((16,), i32),
            pltpu.VMEM((_PER_TILE,), i32),
            pltpu.VMEM((64,), i32),
            pltpu.VMEM((64,), f32),
            pltpu.VMEM((12, 128), i32),
            pltpu.VMEM((12, 128), f32),
            pltpu.VMEM((16 * _C,), f32),
            pltpu.VMEM((2 * _PER_TILE,), f32),
            pltpu.VMEM((_PER_TILE,), f32),
            pltpu.VMEM_SHARED((256,), i32),
            pltpu.VMEM_SHARED((16 * _SLOT,), i32),
            pltpu.VMEM_SHARED((16 * _SLOT,), i32),
            pltpu.VMEM_SHARED((2 * _PUB,), i32),
            pltpu.SemaphoreType.DMA,
            pltpu.SemaphoreType.DMA,
        ],
    )
    return kern(nms_flat, scores_flat, desc_flat)


def kernel(scores_map, descriptor_map):
    b, _, h, w = scores_map.shape
    nms = _nms_map(scores_map).reshape(b * h * w)
    kp, de, ks = _sc_select(
        nms,
        scores_map.reshape(b * h * w),
        descriptor_map.reshape(b * _C * h * w),
    )
    keypoints = kp.reshape(b, 2 * _KPAD)[:, : 2 * _TOP_K].reshape(b, _TOP_K, 2)
    descriptors = de.reshape(b, _KPAD * _C)[:, : _TOP_K * _C].reshape(
        b, _TOP_K, _C)
    kptscores = ks.reshape(b, _KPAD)[:, :_TOP_K]
    return keypoints, descriptors, kptscores


# speedup vs baseline: 1.0336x; 1.0336x over previous
import functools

import jax
import jax.numpy as jnp
from jax import lax
from jax.experimental import pallas as pl
from jax.experimental.pallas import tpu as pltpu
from jax.experimental.pallas import tpu_sc as plsc

_RADIUS = 2
_TOP_K = 5000
_NEG = -jnp.inf

_H = 384
_W = 384
_HW = _H * _W
_B = 4
_C = 96
_SPAN = _HW // 8
_SLOT = 2176
_NSLOT = 8 * _SLOT
_SENT = -1
_PUB = 5120
_PER_TILE = 640
_KPAD = 8 * _PER_TILE
_NCH = _PER_TILE // 16


def _mp5(x):
    h, w = x.shape
    negr = jnp.full((2, w), _NEG, x.dtype)
    xp = jnp.concatenate([negr, x, negr], axis=0)
    r = jnp.maximum(
        jnp.maximum(xp[0:h], xp[1 : h + 1]),
        jnp.maximum(xp[3 : h + 3], xp[4 : h + 4]),
    )
    r = jnp.maximum(r, x)
    negc = jnp.full((h, 2), _NEG, x.dtype)
    rp = jnp.concatenate([negc, r, negc], axis=1)
    out = jnp.maximum(
        jnp.maximum(rp[:, 0:w], rp[:, 1 : w + 1]),
        jnp.maximum(rp[:, 3 : w + 3], rp[:, 4 : w + 4]),
    )
    return jnp.maximum(out, r)


def _nms_body(s_ref, o_ref):
    scores = s_ref[0, 0]
    h, w = scores.shape
    zeros = jnp.zeros_like(scores)
    max_mask = scores == _mp5(scores)
    for _ in range(2):
        supp_mask = _mp5(max_mask.astype(scores.dtype)) > 0.0
        supp_scores = jnp.where(supp_mask, zeros, scores)
        new_max_mask = supp_scores == _mp5(supp_scores)
        max_mask = max_mask | (new_max_mask & (~supp_mask))
    nms = jnp.where(max_mask, scores, zeros)
    r = _RADIUS
    row = lax.broadcasted_iota(jnp.int32, (h, w), 0)
    col = lax.broadcasted_iota(jnp.int32, (h, w), 1)
    interior = (row > r) & (row < h - r) & (col > r) & (col < w - r)
    o_ref[0, 0] = jnp.where(interior, nms, zeros)


def _nms_map(scores_map):
    b, c, h, w = scores_map.shape
    return pl.pallas_call(
        _nms_body,
        out_shape=jax.ShapeDtypeStruct((b, c, h, w), scores_map.dtype),
        grid=(b,),
        in_specs=[pl.BlockSpec((1, 1, h, w), lambda i: (i, 0, 0, 0))],
        out_specs=pl.BlockSpec((1, 1, h, w), lambda i: (i, 0, 0, 0)),
    )(scores_map)


def _iota16():
    return lax.iota(jnp.int32, 16)


_GTR_DNUMS = lax.GatherDimensionNumbers(
    offset_dims=(), collapsed_slice_dims=(0,), start_index_map=(0,))


def _vperm(x, idx):
    return lax.gather(
        x, idx[:, None], dimension_numbers=_GTR_DNUMS, slice_sizes=(1,),
        mode=lax.GatherScatterMode.PROMISE_IN_BOUNDS)


def _dup_rank(d, m, rtmp):
    del rtmp
    it = _iota16()
    d2 = jnp.where(m, d, 2048 + it)
    rank = jnp.zeros((16,), jnp.int32)
    for s in range(1, 16):
        shifted = _vperm(d2, jnp.maximum(it - s, 0))
        hit = (shifted == d2) & (it >= s)
        rank = rank + jnp.where(hit, 1, 0)
    return rank


def _radix_sort(sv, si, ov, oi, hist, rtmp, n_live):
    it = _iota16()
    zeros16 = jnp.zeros((16,), jnp.int32)
    ones16 = jnp.full((16,), 1, jnp.int32)

    def _pass(p, _):
        shift = jnp.where(p == 0, 0, jnp.where(p == 1, 11, 22))
        limit = jnp.where(p == 0, _NSLOT, n_live)

        def _clr(i, __):
            hist[pl.ds(i * 16, 16)] = zeros16
            return 0

        lax.fori_loop(0, 2048 // 16, _clr, 0)

        def _count(i, __):
            k = sv[pl.ds(i * 16, 16)]
            m = (k != _SENT) & ((i * 16 + it) < limit)

            @pl.when(jnp.any(m))
            def __():
                d = lax.shift_right_logical(k, shift) & 2047
                plsc.addupdate_scatter(hist, [d], ones16, mask=m)
            return 0

        lax.fori_loop(0, _NSLOT // 16, _count, 0)

        def _prefix(i, carry):
            c = hist[pl.ds(i * 16, 16)]
            incl = plsc.cumsum(c)
            hist[pl.ds(i * 16, 16)] = carry + (incl - c)
            return carry + incl[15]

        lax.fori_loop(0, 2048 // 16, _prefix, jnp.int32(0))

        def _scatter(i, __):
            k = sv[pl.ds(i * 16, 16)]
            m = (k != _SENT) & ((i * 16 + it) < limit)

            @pl.when(jnp.any(m))
            def __():
                v = si[pl.ds(i * 16, 16)]
                d = lax.shift_right_logical(k, shift) & 2047
                base = plsc.load_gather(hist, [d])
                pos = base + _dup_rank(d, m, rtmp)
                plsc.store_scatter(ov, [pos], k, mask=m)
                plsc.store_scatter(oi, [pos], v, mask=m)
                plsc.addupdate_scatter(hist, [d], ones16, mask=m)
            return 0

        lax.fori_loop(0, _NSLOT // 16, _scatter, 0)

        def _copyback(i, __):
            @pl.when(i * 16 < n_live)
            def __():
                sv[pl.ds(i * 16, 16)] = ov[pl.ds(i * 16, 16)]
                si[pl.ds(i * 16, 16)] = oi[pl.ds(i * 16, 16)]
            return 0

        lax.fori_loop(0, _NSLOT // 16, _copyback, 0)
        return 0

    lax.fori_loop(0, 3, _pass, 0)


def _newton_rsqrt(x):
    bits = lax.bitcast_convert_type(x, jnp.int32)
    y = lax.bitcast_convert_type(
        jnp.int32(0x5F3759DF) - lax.shift_right_logical(bits, 1), jnp.float32)
    for _ in range(3):
        y = y * (1.5 - 0.5 * x * y * y)
    return y


def _sc_body(nms_hbm, sco_hbm, des_hbm, kp_out, de_out, ks_out,
             nmsbuf, lock, loci, cntbuf, cnt_all,
             sv, si, ov, oi, hist, rtmp, sidx,
             scidx, sdst, didx, gbuf, outb, kpbuf, ksbuf,
             counts_sh, ck_sh, ci_sh, sidx_sh, sem, sem2):
    cid = lax.axis_index("c")
    sid = lax.axis_index("s")
    bl = sid // 8
    chunk = sid % 8
    batch = 2 * cid + bl
    it = _iota16()

    pltpu.sync_copy(nms_hbm.at[pl.ds(batch * _HW + chunk * _SPAN, _SPAN)],
                    nmsbuf)
    base_idx = chunk * _SPAN

    def _compact(i, ptr):
        v = nmsbuf[pl.ds(i * 16, 16)]
        m = (v > 0.0) & (ptr < _SLOT - 16)
        kv = ~lax.bitcast_convert_type(v, jnp.int32)
        idxv = base_idx + i * 16 + it
        mi = m.astype(jnp.int32)
        pref = plsc.cumsum(mi)
        pos = ptr + (pref - mi)
        plsc.store_scatter(lock, [pos], kv, mask=m)
        plsc.store_scatter(loci, [pos], idxv, mask=m)
        return ptr + pref[15]

    ptr = lax.fori_loop(0, _SPAN // 16, _compact, jnp.int32(0))

    def _pad(i, _):
        cur = lock[pl.ds(i * 16, 16)]
        m = (i * 16 + it) < ptr
        lock[pl.ds(i * 16, 16)] = jnp.where(m, cur, _SENT)
        return 0

    lax.fori_loop(0, _SLOT // 16, _pad, 0)

    slot = (bl * 8 + chunk) * _SLOT
    pltpu.sync_copy(lock.at[pl.ds(0, _SLOT)], ck_sh.at[pl.ds(slot, _SLOT)])
    pltpu.sync_copy(loci.at[pl.ds(0, _SLOT)], ci_sh.at[pl.ds(slot, _SLOT)])
    cntbuf[...] = jnp.broadcast_to(ptr, (16,))
    pltpu.sync_copy(cntbuf, counts_sh.at[pl.ds(sid * 16, 16)])
    plsc.subcore_barrier()

    pltpu.sync_copy(counts_sh, cnt_all)
    nv = jnp.zeros((16,), jnp.int32)
    for j in range(8):
        nv = nv + cnt_all[pl.ds((bl * 8 + j) * 16, 16)]
    n_live = nv[0]

    @pl.when(chunk == 0)
    def _sort():
        for j in range(8):
            src = (bl * 8 + j) * _SLOT
            pltpu.sync_copy(ck_sh.at[pl.ds(src, _SLOT)],
                            sv.at[pl.ds(j * _SLOT, _SLOT)])
            pltpu.sync_copy(ci_sh.at[pl.ds(src, _SLOT)],
                            si.at[pl.ds(j * _SLOT, _SLOT)])
        _radix_sort(sv, si, ov, oi, hist, rtmp, n_live)
        pltpu.sync_copy(si.at[pl.ds(0, _PUB)],
                        sidx_sh.at[pl.ds(bl * _PUB, _PUB)])

    plsc.subcore_barrier()

    myoff = chunk * _PER_TILE
    pltpu.sync_copy(sidx_sh.at[pl.ds(bl * _PUB + myoff, _PER_TILE)], sidx)

    kp_pos0 = 2 * it
    d_pos = it * _C

    def _emit(j, _):
        p = myoff + j * 16 + it
        sidx_v = sidx[pl.ds(j * 16, 16)]
        real = p < n_live
        idx = jnp.where(real, sidx_v, p - n_live)

        t = lax.shift_right_logical(idx, 7)
        ky = lax.shift_right_logical(t * 683, 11)
        kx = idx - ky * 384
        kp_x = kx.astype(jnp.float32)
        kp_y = ky.astype(jnp.float32)
        ax = kp_x / 383.0 * 2.0 - 1.0
        ay = kp_y / 383.0 * 2.0 - 1.0
        plsc.store_scatter(kpbuf, [kp_pos0 + j * 32], ax)
        plsc.store_scatter(kpbuf, [kp_pos0 + j * 32 + 1], ay)

        px = (ax + 1.0) / 2.0 * 383.0
        py = (ay + 1.0) / 2.0 * 383.0
        x0i0 = px.astype(jnp.int32)
        y0i0 = py.astype(jnp.int32)
        x0 = x0i0.astype(jnp.float32)
        y0 = y0i0.astype(jnp.float32)
        wx1 = px - x0
        wx0 = 1.0 - wx1
        wy1 = py - y0
        wy0 = 1.0 - wy1
        x0i = jnp.minimum(x0i0, 383)
        x1i = jnp.minimum(x0i0 + 1, 383)
        y0i = jnp.minimum(y0i0, 383)
        y1i = jnp.minimum(y0i0 + 1, 383)
        sbase = batch * _HW
        scidx[pl.ds(0, 16)] = sbase + y0i * 384 + x0i
        scidx[pl.ds(16, 16)] = sbase + y0i * 384 + x1i
        scidx[pl.ds(32, 16)] = sbase + y1i * 384 + x0i
        scidx[pl.ds(48, 16)] = sbase + y1i * 384 + x1i
        sc_cp = pltpu.async_copy(sco_hbm.at[scidx], sdst, sem2)

        kxi = px.astype(jnp.int32)
        kyi = py.astype(jnp.int32)
        dflat = batch * (_C * _HW) + kyi * 384 + kxi
        for r in range(12):
            for q in range(8):
                c = r * 8 + q
                didx[r, pl.ds(q * 16, 16)] = dflat + c * _HW
        copies = [pltpu.async_copy(des_hbm.at[didx.at[r]], gbuf.at[r], sem)
                  for r in range(12)]

        sc_cp.wait()
        v00 = sdst[pl.ds(0, 16)]
        v01 = sdst[pl.ds(16, 16)]
        v10 = sdst[pl.ds(32, 16)]
        v11 = sdst[pl.ds(48, 16)]
        ksbuf[pl.ds(j * 16, 16)] = (v00 * wy0 * wx0 + v01 * wy0 * wx1
                                    + v10 * wy1 * wx0 + v11 * wy1 * wx1)

        for cp in copies:
            cp.wait()
        acc = jnp.zeros((16,), jnp.float32)
        for r in range(12):
            for q in range(8):
                g = gbuf[r, pl.ds(q * 16, 16)]
                acc = acc + g * g
        rns = _newton_rsqrt(acc)
        for r in range(12):
            for q in range(8):
                c = r * 8 + q
                g = gbuf[r, pl.ds(q * 16, 16)]
                plsc.store_scatter(outb, [d_pos + c], g * rns)
        pltpu.sync_copy(
            outb,
            de_out.at[pl.ds(batch * (_KPAD * _C) + (myoff + j * 16) * _C,
                            16 * _C)])
        return 0

    lax.fori_loop(0, _NCH, _emit, 0)

    pltpu.sync_copy(
        kpbuf,
        kp_out.at[pl.ds(batch * 2 * _KPAD + 2 * myoff, 2 * _PER_TILE)])
    pltpu.sync_copy(
        ksbuf, ks_out.at[pl.ds(batch * _KPAD + myoff, _PER_TILE)])


def _sc_select(nms_flat, scores_flat, desc_flat):
    mesh = plsc.VectorSubcoreMesh(core_axis_name="c", subcore_axis_name="s")
    f32 = jnp.float32
    i32 = jnp.int32
    kern = pl.kernel(
        _sc_body,
        out_type=[
            jax.ShapeDtypeStruct((_B * 2 * _KPAD,), f32),
            jax.ShapeDtypeStruct((_B * _KPAD * _C,), f32),
            jax.ShapeDtypeStruct((_B * _KPAD,), f32),
        ],
        mesh=mesh,
        compiler_params=pltpu.CompilerParams(needs_layout_passes=False),
        scratch_types=[
            pltpu.VMEM((_SPAN,), f32),
            pltpu.VMEM((_SLOT + 16,), i32),
            pltpu.VMEM((_SLOT + 16,), i32),
            pltpu.VMEM((16,), i32),
            pltpu.VMEM((256,), i32),
            pltpu.VMEM((_NSLOT,), i32),
            pltpu.VMEM((_NSLOT,), i32),
            pltpu.VMEM((_NSLOT,), i32),
            pltpu.VMEM((_NSLOT,), i32),
            pltpu.VMEM((2048,), i32),
            pltpu.VMEM((16,), i32),
            pltpu.VMEM((_PER_TILE,), i32),
            pltpu.VMEM((64,), i32),
            pltpu.VMEM((64,), f32),
            pltpu.VMEM((12, 128), i32),
            pltpu.VMEM((12, 128), f32),
            pltpu.VMEM((16 * _C,), f32),
            pltpu.VMEM((2 * _PER_TILE,), f32),
            pltpu.VMEM((_PER_TILE,), f32),
            pltpu.VMEM_SHARED((256,), i32),
            pltpu.VMEM_SHARED((16 * _SLOT,), i32),
            pltpu.VMEM_SHARED((16 * _SLOT,), i32),
            pltpu.VMEM_SHARED((2 * _PUB,), i32),
            pltpu.SemaphoreType.DMA,
            pltpu.SemaphoreType.DMA,
        ],
    )
    return kern(nms_flat, scores_flat, desc_flat)


def kernel(scores_map, descriptor_map):
    b, _, h, w = scores_map.shape
    nms = _nms_map(scores_map).reshape(b * h * w)
    kp, de, ks = _sc_select(
        nms,
        scores_map.reshape(b * h * w),
        descriptor_map.reshape(b * _C * h * w),
    )
    keypoints = kp.reshape(b, 2 * _KPAD)[:, : 2 * _TOP_K].reshape(b, _TOP_K, 2)
    descriptors = de.reshape(b, _KPAD * _C)[:, : _TOP_K * _C].reshape(
        b, _TOP_K, _C)
    kptscores = ks.reshape(b, _KPAD)[:, :_TOP_K]
    return keypoints, descriptors, kptscores


# speedup vs baseline: 1.4729x; 1.4250x over previous
import functools

import jax
import jax.numpy as jnp
from jax import lax
from jax.experimental import pallas as pl
from jax.experimental.pallas import tpu as pltpu
from jax.experimental.pallas import tpu_sc as plsc

_RADIUS = 2
_TOP_K = 5000
_NEG = -jnp.inf

_H = 384
_W = 384
_HW = _H * _W
_B = 4
_C = 96
_SPAN = _HW // 8
_SLOT = 2176
_NSLOT = 8 * _SLOT
_SENT = -1
_PUB = 5120
_PER_TILE = 640
_KPAD = 8 * _PER_TILE
_NCH = _PER_TILE // 16


def _mp5(x):
    h, w = x.shape
    negr = jnp.full((2, w), _NEG, x.dtype)
    xp = jnp.concatenate([negr, x, negr], axis=0)
    r = jnp.maximum(
        jnp.maximum(xp[0:h], xp[1 : h + 1]),
        jnp.maximum(xp[3 : h + 3], xp[4 : h + 4]),
    )
    r = jnp.maximum(r, x)
    negc = jnp.full((h, 2), _NEG, x.dtype)
    rp = jnp.concatenate([negc, r, negc], axis=1)
    out = jnp.maximum(
        jnp.maximum(rp[:, 0:w], rp[:, 1 : w + 1]),
        jnp.maximum(rp[:, 3 : w + 3], rp[:, 4 : w + 4]),
    )
    return jnp.maximum(out, r)


def _nms_body(s_ref, o_ref):
    scores = s_ref[0, 0]
    h, w = scores.shape
    zeros = jnp.zeros_like(scores)
    max_mask = scores == _mp5(scores)
    for _ in range(2):
        supp_mask = _mp5(max_mask.astype(scores.dtype)) > 0.0
        supp_scores = jnp.where(supp_mask, zeros, scores)
        new_max_mask = supp_scores == _mp5(supp_scores)
        max_mask = max_mask | (new_max_mask & (~supp_mask))
    nms = jnp.where(max_mask, scores, zeros)
    r = _RADIUS
    row = lax.broadcasted_iota(jnp.int32, (h, w), 0)
    col = lax.broadcasted_iota(jnp.int32, (h, w), 1)
    interior = (row > r) & (row < h - r) & (col > r) & (col < w - r)
    o_ref[0, 0] = jnp.where(interior, nms, zeros)


def _nms_map(scores_map):
    b, c, h, w = scores_map.shape
    return pl.pallas_call(
        _nms_body,
        out_shape=jax.ShapeDtypeStruct((b, c, h, w), scores_map.dtype),
        grid=(b,),
        in_specs=[pl.BlockSpec((1, 1, h, w), lambda i: (i, 0, 0, 0))],
        out_specs=pl.BlockSpec((1, 1, h, w), lambda i: (i, 0, 0, 0)),
    )(scores_map)


def _iota16():
    return lax.iota(jnp.int32, 16)


_GTR_DNUMS = lax.GatherDimensionNumbers(
    offset_dims=(), collapsed_slice_dims=(0,), start_index_map=(0,))


def _vperm(x, idx):
    return lax.gather(
        x, idx[:, None], dimension_numbers=_GTR_DNUMS, slice_sizes=(1,),
        mode=lax.GatherScatterMode.PROMISE_IN_BOUNDS)


def _dup_rank(d, m):
    it = _iota16()
    d2 = jnp.where(m, d, 2048 + it)
    rank = jnp.zeros((16,), jnp.int32)
    for s in range(1, 16):
        shifted = _vperm(d2, jnp.maximum(it - s, 0))
        hit = (shifted == d2) & (it >= s)
        rank = rank + jnp.where(hit, 1, 0)
    return rank


def _parallel_radix_sort(lock, loci, hist, hists8, hists_sh, dst_k, dst_i,
                         nms_hbm, nmsbuf, sem, n_live, gbase, me_slot,
                         bl, me):
    it = _iota16()
    zeros16 = jnp.zeros((16,), jnp.int32)
    ones16 = jnp.full((16,), 1, jnp.int32)

    def _pass(pnum, _):
        shift = jnp.where(pnum == 0, 0, jnp.where(pnum == 1, 11, 22))
        limit = jnp.where(pnum == 0, _NSLOT, n_live)

        def _clr(i, __):
            hist[pl.ds(i * 16, 16)] = zeros16
            return 0

        lax.fori_loop(0, 2048 // 16, _clr, 0)

        def _count(i, __):
            k = lock[pl.ds(i * 16, 16)]
            m = (k != _SENT) & ((me_slot + i * 16 + it) < limit)
            d = lax.shift_right_logical(k, shift) & 2047
            plsc.addupdate_scatter(hist, [d], ones16, mask=m)
            return 0

        lax.fori_loop(0, _SLOT // 16, _count, 0)

        pltpu.sync_copy(hist, hists_sh.at[pl.ds((bl * 8 + me) * 2048, 2048)])
        plsc.subcore_barrier()
        pltpu.sync_copy(hists_sh.at[pl.ds(bl * 8 * 2048, 8 * 2048)], hists8)

        def _offsets(i, carry):
            g = jnp.zeros((16,), jnp.int32)
            part = jnp.zeros((16,), jnp.int32)
            for t in range(8):
                h_t = hists8[pl.ds(t * 2048 + i * 16, 16)]
                g = g + h_t
                part = part + h_t * (t < me)
            incl = plsc.cumsum(g)
            hist[pl.ds(i * 16, 16)] = carry + (incl - g) + part
            return carry + incl[15]

        lax.fori_loop(0, 2048 // 16, _offsets, jnp.int32(0))

        def _seg(sg, __):
            def _scatter(q, ___):
                c = sg * 17 + q
                k = lock[pl.ds(c * 16, 16)]
                m = (k != _SENT) & ((me_slot + c * 16 + it) < limit)
                d = lax.shift_right_logical(k, shift) & 2047
                base = plsc.load_gather(hist, [d])
                pos = base + _dup_rank(d, m)
                posg = jnp.where(m, gbase + pos, 2 * _NSLOT + it)
                plsc.addupdate_scatter(hist, [d], ones16, mask=m)
                pltpu.async_copy(lock.at[pl.ds(c * 16, 16)],
                                 dst_k.at[posg], sem)
                pltpu.async_copy(loci.at[pl.ds(c * 16, 16)],
                                 dst_i.at[posg], sem)
                return 0

            lax.fori_loop(0, 17, _scatter, 0)
            pltpu.make_async_copy(
                nms_hbm.at[pl.ds(0, 17 * 32)], nmsbuf.at[pl.ds(0, 17 * 32)],
                sem).wait()
            return 0

        lax.fori_loop(0, 8, _seg, 0)
        plsc.subcore_barrier()
        pltpu.sync_copy(dst_k.at[pl.ds(gbase + me_slot, _SLOT)],
                        lock.at[pl.ds(0, _SLOT)])
        pltpu.sync_copy(dst_i.at[pl.ds(gbase + me_slot, _SLOT)],
                        loci.at[pl.ds(0, _SLOT)])
        plsc.subcore_barrier()
        return 0

    lax.fori_loop(0, 3, _pass, 0)


def _newton_rsqrt(x):
    bits = lax.bitcast_convert_type(x, jnp.int32)
    y = lax.bitcast_convert_type(
        jnp.int32(0x5F3759DF) - lax.shift_right_logical(bits, 1), jnp.float32)
    for _ in range(3):
        y = y * (1.5 - 0.5 * x * y * y)
    return y


def _sc_body(nms_hbm, sco_hbm, des_hbm, kp_out, de_out, ks_out,
             nmsbuf, lock, loci, cntbuf, cnt_all,
             hist, hists8, sidx,
             scidx, sdst, didx, gbuf, outb, kpbuf, ksbuf,
             counts_sh, hists_sh, s0k, s0i, sem, sem2):
    cid = lax.axis_index("c")
    sid = lax.axis_index("s")
    bl = sid // 8
    chunk = sid % 8
    batch = 2 * cid + bl
    it = _iota16()

    pltpu.sync_copy(nms_hbm.at[pl.ds(batch * _HW + chunk * _SPAN, _SPAN)],
                    nmsbuf)
    base_idx = chunk * _SPAN

    def _compact(i, ptr):
        v = nmsbuf[pl.ds(i * 16, 16)]
        m = (v > 0.0) & (ptr < _SLOT - 16)
        kv = ~lax.bitcast_convert_type(v, jnp.int32)
        idxv = base_idx + i * 16 + it
        mi = m.astype(jnp.int32)
        pref = plsc.cumsum(mi)
        pos = ptr + (pref - mi)
        plsc.store_scatter(lock, [pos], kv, mask=m)
        plsc.store_scatter(loci, [pos], idxv, mask=m)
        return ptr + pref[15]

    ptr = lax.fori_loop(0, _SPAN // 16, _compact, jnp.int32(0))

    def _pad(i, _):
        cur = lock[pl.ds(i * 16, 16)]
        m = (i * 16 + it) < ptr
        lock[pl.ds(i * 16, 16)] = jnp.where(m, cur, _SENT)
        return 0

    lax.fori_loop(0, _SLOT // 16, _pad, 0)

    cntbuf[...] = jnp.broadcast_to(ptr, (16,))
    pltpu.sync_copy(cntbuf, counts_sh.at[pl.ds(sid * 16, 16)])
    plsc.subcore_barrier()

    pltpu.sync_copy(counts_sh, cnt_all)
    nv = jnp.zeros((16,), jnp.int32)
    for j in range(8):
        nv = nv + cnt_all[pl.ds((bl * 8 + j) * 16, 16)]
    n_live = nv[0]

    me = chunk
    gbase = bl * _NSLOT
    me_slot = me * _SLOT
    _parallel_radix_sort(lock, loci, hist, hists8, hists_sh, s0k, s0i,
                         nms_hbm, nmsbuf, sem, n_live, gbase, me_slot,
                         bl, me)

    myoff = chunk * _PER_TILE
    pltpu.sync_copy(s0i.at[pl.ds(gbase + myoff, _PER_TILE)], sidx)

    kp_pos0 = 2 * it
    d_pos = it * _C

    def _emit(j, _):
        p = myoff + j * 16 + it
        sidx_v = sidx[pl.ds(j * 16, 16)]
        real = p < n_live
        idx = jnp.where(real, sidx_v, p - n_live)

        t = lax.shift_right_logical(idx, 7)
        ky = lax.shift_right_logical(t * 683, 11)
        kx = idx - ky * 384
        kp_x = kx.astype(jnp.float32)
        kp_y = ky.astype(jnp.float32)
        ax = kp_x / 383.0 * 2.0 - 1.0
        ay = kp_y / 383.0 * 2.0 - 1.0
        plsc.store_scatter(kpbuf, [kp_pos0 + j * 32], ax)
        plsc.store_scatter(kpbuf, [kp_pos0 + j * 32 + 1], ay)

        px = (ax + 1.0) / 2.0 * 383.0
        py = (ay + 1.0) / 2.0 * 383.0
        x0i0 = px.astype(jnp.int32)
        y0i0 = py.astype(jnp.int32)
        x0 = x0i0.astype(jnp.float32)
        y0 = y0i0.astype(jnp.float32)
        wx1 = px - x0
        wx0 = 1.0 - wx1
        wy1 = py - y0
        wy0 = 1.0 - wy1
        x0i = jnp.minimum(x0i0, 383)
        x1i = jnp.minimum(x0i0 + 1, 383)
        y0i = jnp.minimum(y0i0, 383)
        y1i = jnp.minimum(y0i0 + 1, 383)
        sbase = batch * _HW
        scidx[pl.ds(0, 16)] = sbase + y0i * 384 + x0i
        scidx[pl.ds(16, 16)] = sbase + y0i * 384 + x1i
        scidx[pl.ds(32, 16)] = sbase + y1i * 384 + x0i
        scidx[pl.ds(48, 16)] = sbase + y1i * 384 + x1i
        sc_cp = pltpu.async_copy(sco_hbm.at[scidx], sdst, sem2)

        kxi = px.astype(jnp.int32)
        kyi = py.astype(jnp.int32)
        dflat = batch * (_C * _HW) + kyi * 384 + kxi
        for r in range(12):
            for q in range(8):
                c = r * 8 + q
                didx[r, pl.ds(q * 16, 16)] = dflat + c * _HW
        copies = [pltpu.async_copy(des_hbm.at[didx.at[r]], gbuf.at[r], sem)
                  for r in range(12)]

        sc_cp.wait()
        v00 = sdst[pl.ds(0, 16)]
        v01 = sdst[pl.ds(16, 16)]
        v10 = sdst[pl.ds(32, 16)]
        v11 = sdst[pl.ds(48, 16)]
        ksbuf[pl.ds(j * 16, 16)] = (v00 * wy0 * wx0 + v01 * wy0 * wx1
                                    + v10 * wy1 * wx0 + v11 * wy1 * wx1)

        for cp in copies:
            cp.wait()
        acc = jnp.zeros((16,), jnp.float32)
        for r in range(12):
            for q in range(8):
                g = gbuf[r, pl.ds(q * 16, 16)]
                acc = acc + g * g
        rns = _newton_rsqrt(acc)
        for r in range(12):
            for q in range(8):
                c = r * 8 + q
                g = gbuf[r, pl.ds(q * 16, 16)]
                plsc.store_scatter(outb, [d_pos + c], g * rns)
        pltpu.sync_copy(
            outb,
            de_out.at[pl.ds(batch * (_KPAD * _C) + (myoff + j * 16) * _C,
                            16 * _C)])
        return 0

    lax.fori_loop(0, _NCH, _emit, 0)

    pltpu.sync_copy(
        kpbuf,
        kp_out.at[pl.ds(batch * 2 * _KPAD + 2 * myoff, 2 * _PER_TILE)])
    pltpu.sync_copy(
        ksbuf, ks_out.at[pl.ds(batch * _KPAD + myoff, _PER_TILE)])


def _sc_select(nms_flat, scores_flat, desc_flat):
    mesh = plsc.VectorSubcoreMesh(core_axis_name="c", subcore_axis_name="s")
    f32 = jnp.float32
    i32 = jnp.int32
    kern = pl.kernel(
        _sc_body,
        out_type=[
            jax.ShapeDtypeStruct((_B * 2 * _KPAD,), f32),
            jax.ShapeDtypeStruct((_B * _KPAD * _C,), f32),
            jax.ShapeDtypeStruct((_B * _KPAD,), f32),
        ],
        mesh=mesh,
        compiler_params=pltpu.CompilerParams(needs_layout_passes=False),
        scratch_types=[
            pltpu.VMEM((_SPAN,), f32),
            pltpu.VMEM((_SLOT + 16,), i32),
            pltpu.VMEM((_SLOT + 16,), i32),
            pltpu.VMEM((16,), i32),
            pltpu.VMEM((256,), i32),
            pltpu.VMEM((2048,), i32),
            pltpu.VMEM((8 * 2048,), i32),
            pltpu.VMEM((_PER_TILE,), i32),
            pltpu.VMEM((64,), i32),
            pltpu.VMEM((64,), f32),
            pltpu.VMEM((12, 128), i32),
            pltpu.VMEM((12, 128), f32),
            pltpu.VMEM((16 * _C,), f32),
            pltpu.VMEM((2 * _PER_TILE,), f32),
            pltpu.VMEM((_PER_TILE,), f32),
            pltpu.VMEM_SHARED((256,), i32),
            pltpu.VMEM_SHARED((2 * 8 * 2048,), i32),
            pltpu.VMEM_SHARED((2 * _NSLOT + 16,), i32),
            pltpu.VMEM_SHARED((2 * _NSLOT + 16,), i32),
            pltpu.SemaphoreType.DMA,
            pltpu.SemaphoreType.DMA,
        ],
    )
    return kern(nms_flat, scores_flat, desc_flat)


def kernel(scores_map, descriptor_map):
    b, _, h, w = scores_map.shape
    nms = _nms_map(scores_map).reshape(b * h * w)
    kp, de, ks = _sc_select(
        nms,
        scores_map.reshape(b * h * w),
        descriptor_map.reshape(b * _C * h * w),
    )
    keypoints = kp.reshape(b, 2 * _KPAD)[:, : 2 * _TOP_K].reshape(b, _TOP_K, 2)
    descriptors = de.reshape(b, _KPAD * _C)[:, : _TOP_K * _C].reshape(
        b, _TOP_K, _C)
    kptscores = ks.reshape(b, _KPAD)[:, :_TOP_K]
    return keypoints, descriptors, kptscores
